# baseline (speedup 1.0000x reference)
import functools
import numpy as np
import jax
import jax.numpy as jnp
from jax import lax
from jax.experimental import pallas as pl
from jax.experimental.pallas import tpu as pltpu
from jax.experimental.pallas import tpu_sc as plsc


def _enc_table(S, E):
    pos = np.arange(S, dtype=np.float64)[:, None]
    i = np.arange(E, dtype=np.float64)[None, :]
    angle = pos / np.power(10000.0, (i - np.mod(i, 2)) / E)
    enc = np.array(angle)
    enc[:, 0::2] = np.sin(angle[:, 0::2])
    enc[:, 1::2] = np.cos(angle[:, 1::2])
    return jnp.asarray(enc, dtype=jnp.float32)


_NC, _NS, _L = 2, 16, 16
_NW = _NC * _NS


@functools.partial(jax.jit, static_argnums=(2, 3, 4))
def _sc_call(xf, encf, B, S, E):
    R = 8
    CHUNK = S // _NW
    NT = CHUNK // R
    RE = R * E

    mesh = plsc.VectorSubcoreMesh(core_axis_name="c", subcore_axis_name="s")

    @functools.partial(
        pl.kernel,
        out_type=jax.ShapeDtypeStruct((B, S * E), jnp.float32),
        mesh=mesh,
        scratch_types=[
            pltpu.VMEM((RE,), jnp.float32),
            pltpu.VMEM((B, RE), jnp.float32),
            pltpu.VMEM((B, RE), jnp.float32),
        ],
    )
    def sc_kernel(x_hbm, enc_hbm, out_hbm, enc_v, xs_v, os_v):
        wid = lax.axis_index("s") * _NC + lax.axis_index("c")
        base = wid * CHUNK * E

        def tile_body(t, carry):
            off = base + t * RE
            pltpu.sync_copy(enc_hbm.at[pl.ds(off, RE)], enc_v)
            for b in range(B):
                pltpu.sync_copy(x_hbm.at[b, pl.ds(off, RE)], xs_v.at[b])

            def chunk(j, c2):
                o = j * _L
                e = enc_v[pl.ds(o, _L)]
                for b in range(B):
                    xv = xs_v[b, pl.ds(o, _L)]
                    os_v[b, pl.ds(o, _L)] = jnp.where(xv == 0.0, xv, e)
                return c2

            lax.fori_loop(0, RE // _L, chunk, 0)
            for b in range(B):
                pltpu.sync_copy(os_v.at[b], out_hbm.at[b, pl.ds(off, RE)])
            return carry

        lax.fori_loop(0, NT, tile_body, 0)

    return sc_kernel(xf, encf)


def kernel(x):
    B, S, E = x.shape
    enc = _enc_table(S, E).reshape(S * E)
    out = _sc_call(x.reshape(B, S * E), enc, B, S, E)
    return out.reshape(B, S, E)


# speedup vs baseline: 1.5035x; 1.5035x over previous
import functools
import numpy as np
import jax
import jax.numpy as jnp
from jax import lax
from jax.experimental import pallas as pl
from jax.experimental.pallas import tpu as pltpu
from jax.experimental.pallas import tpu_sc as plsc


def _enc_table(S, E):
    pos = np.arange(S, dtype=np.float64)[:, None]
    i = np.arange(E, dtype=np.float64)[None, :]
    angle = pos / np.power(10000.0, (i - np.mod(i, 2)) / E)
    enc = np.array(angle)
    enc[:, 0::2] = np.sin(angle[:, 0::2])
    enc[:, 1::2] = np.cos(angle[:, 1::2])
    return jnp.asarray(enc, dtype=jnp.float32)


_NC, _NS, _L = 2, 16, 16
_NW = _NC * _NS


@functools.partial(jax.jit, static_argnums=(2, 3, 4))
def _sc_call(xf, encf, B, S, E):
    R = 8
    CHUNK = S // _NW
    NT = CHUNK // R
    RE = R * E

    mesh = plsc.VectorSubcoreMesh(core_axis_name="c", subcore_axis_name="s")

    @functools.partial(
        pl.kernel,
        out_type=jax.ShapeDtypeStruct((B, S * E), jnp.float32),
        mesh=mesh,
        scratch_types=[
            pltpu.VMEM((2, RE), jnp.float32),
            pltpu.VMEM((2, B, RE), jnp.float32),
            pltpu.SemaphoreType.DMA,
            pltpu.SemaphoreType.DMA,
            pltpu.SemaphoreType.DMA,
            pltpu.SemaphoreType.DMA,
        ],
    )
    def sc_kernel(x_hbm, enc_hbm, out_hbm, enc_v, xs_v, ld0, ld1, st0, st1):
        wid = lax.axis_index("s") * _NC + lax.axis_index("c")
        base = wid * CHUNK * E
        ld = (ld0, ld1)
        st = (st0, st1)

        def start_load(t):
            p = t % 2
            off = base + t * RE
            pltpu.async_copy(enc_hbm.at[pl.ds(off, RE)], enc_v.at[p], ld[p])
            pltpu.async_copy(x_hbm.at[:, pl.ds(off, RE)], xs_v.at[p], ld[p])

        def wait_load(t):
            p = t % 2
            off = base + t * RE
            pltpu.make_async_copy(
                enc_hbm.at[pl.ds(off, RE)], enc_v.at[p], ld[p]).wait()
            pltpu.make_async_copy(
                x_hbm.at[:, pl.ds(off, RE)], xs_v.at[p], ld[p]).wait()

        def start_store(t):
            p = t % 2
            off = base + t * RE
            pltpu.async_copy(xs_v.at[p], out_hbm.at[:, pl.ds(off, RE)], st[p])

        def wait_store(t):
            p = t % 2
            off = base + t * RE
            pltpu.make_async_copy(
                xs_v.at[p], out_hbm.at[:, pl.ds(off, RE)], st[p]).wait()

        def compute(t):
            p = t % 2

            @plsc.parallel_loop(0, RE, step=_L, unroll=4)
            def _(o):
                e = enc_v[p, pl.ds(o, _L)]
                for b in range(B):
                    xv = xs_v[p, b, pl.ds(o, _L)]
                    xs_v[p, b, pl.ds(o, _L)] = jnp.where(xv == 0.0, xv, e)

        start_load(0)
        for t in range(NT):
            if t + 1 < NT:
                if t >= 1:
                    wait_store(t - 1)
                start_load(t + 1)
            wait_load(t)
            compute(t)
            start_store(t)
        wait_store(NT - 2)
        wait_store(NT - 1)

    return sc_kernel(xf, encf)


def kernel(x):
    B, S, E = x.shape
    enc = _enc_table(S, E).reshape(S * E)
    out = _sc_call(x.reshape(B, S * E), enc, B, S, E)
    return out.reshape(B, S, E)


# speedup vs baseline: 3.2399x; 2.1549x over previous
import functools
import numpy as np
import jax
import jax.numpy as jnp
from jax import lax
from jax.experimental import pallas as pl
from jax.experimental.pallas import tpu as pltpu
from jax.experimental.pallas import tpu_sc as plsc


def _enc_table(S, E):
    pos = np.arange(S, dtype=np.float64)[:, None]
    i = np.arange(E, dtype=np.float64)[None, :]
    angle = pos / np.power(10000.0, (i - np.mod(i, 2)) / E)
    enc = np.array(angle)
    enc[:, 0::2] = np.sin(angle[:, 0::2])
    enc[:, 1::2] = np.cos(angle[:, 1::2])
    return jnp.asarray(enc, dtype=jnp.float32)


_NC, _NS, _L = 2, 16, 16
_NW = _NC * _NS


@functools.partial(jax.jit, static_argnums=(2, 3, 4))
def _sc_call(x, enc, B, S, E):
    R = 8
    CHUNK = S // _NW
    NT = CHUNK // R
    RE = R * E

    mesh = plsc.VectorSubcoreMesh(core_axis_name="c", subcore_axis_name="s")

    @functools.partial(
        pl.kernel,
        out_type=jax.ShapeDtypeStruct((B, S, E), jnp.float32),
        mesh=mesh,
        scratch_types=[
            pltpu.VMEM((2, R, E), jnp.float32),
            pltpu.VMEM((2, B, R, E), jnp.float32),
            pltpu.SemaphoreType.DMA,
            pltpu.SemaphoreType.DMA,
            pltpu.SemaphoreType.DMA,
            pltpu.SemaphoreType.DMA,
        ],
        compiler_params=pltpu.CompilerParams(use_tc_tiling_on_sc=True),
    )
    def sc_kernel(x_hbm, enc_hbm, out_hbm, enc_v, xs_v, ld0, ld1, st0, st1):
        wid = lax.axis_index("s") * _NC + lax.axis_index("c")
        base = wid * CHUNK
        ld = (ld0, ld1)
        st = (st0, st1)

        def start_load(t):
            p = t % 2
            r0 = base + t * R
            pltpu.async_copy(enc_hbm.at[pl.ds(r0, R), :], enc_v.at[p], ld[p])
            pltpu.async_copy(x_hbm.at[:, pl.ds(r0, R), :], xs_v.at[p], ld[p])

        def wait_load(t):
            p = t % 2
            r0 = base + t * R
            pltpu.make_async_copy(
                enc_hbm.at[pl.ds(r0, R), :], enc_v.at[p], ld[p]).wait()
            pltpu.make_async_copy(
                x_hbm.at[:, pl.ds(r0, R), :], xs_v.at[p], ld[p]).wait()

        def start_store(t):
            p = t % 2
            r0 = base + t * R
            pltpu.async_copy(xs_v.at[p], out_hbm.at[:, pl.ds(r0, R), :], st[p])

        def wait_store(t):
            p = t % 2
            r0 = base + t * R
            pltpu.make_async_copy(
                xs_v.at[p], out_hbm.at[:, pl.ds(r0, R), :], st[p]).wait()

        def compute(t):
            p = t % 2

            @plsc.parallel_loop(0, RE, step=_L, unroll=4)
            def _(o):
                r = o >> 10
                c = pl.multiple_of(o & (E - 1), _L)
                e = enc_v[p, r, pl.ds(c, _L)]
                for b in range(B):
                    xv = xs_v[p, b, r, pl.ds(c, _L)]
                    xs_v[p, b, r, pl.ds(c, _L)] = jnp.where(xv == 0.0, xv, e)

        start_load(0)
        for t in range(NT):
            if t + 1 < NT:
                if t >= 1:
                    wait_store(t - 1)
                start_load(t + 1)
            wait_load(t)
            compute(t)
            start_store(t)
        wait_store(NT - 2)
        wait_store(NT - 1)

    return sc_kernel(x, enc)


def kernel(x):
    B, S, E = x.shape
    enc = _enc_table(S, E)
    return _sc_call(x, enc, B, S, E)


# speedup vs baseline: 3.2616x; 1.0067x over previous
import functools
import numpy as np
import jax
import jax.numpy as jnp
from jax import lax
from jax.experimental import pallas as pl
from jax.experimental.pallas import tpu as pltpu
from jax.experimental.pallas import tpu_sc as plsc


def _enc_table(S, E):
    pos = np.arange(S, dtype=np.float64)[:, None]
    i = np.arange(E, dtype=np.float64)[None, :]
    angle = pos / np.power(10000.0, (i - np.mod(i, 2)) / E)
    enc = np.array(angle)
    enc[:, 0::2] = np.sin(angle[:, 0::2])
    enc[:, 1::2] = np.cos(angle[:, 1::2])
    return jnp.asarray(enc, dtype=jnp.float32)


_NC, _NS, _L = 2, 16, 16
_NW = _NC * _NS


@functools.partial(jax.jit, static_argnums=(2, 3, 4))
def _sc_call(x, enc, B, S, E):
    R = 8
    CHUNK = S // _NW
    NT = CHUNK // R
    RE = R * E

    mesh = plsc.VectorSubcoreMesh(core_axis_name="c", subcore_axis_name="s")

    @functools.partial(
        pl.kernel,
        out_type=jax.ShapeDtypeStruct((B, S, E), jnp.float32),
        mesh=mesh,
        scratch_types=[
            pltpu.VMEM((2, R, E), jnp.float32),
            pltpu.VMEM((2, B, R, E), jnp.float32),
            pltpu.SemaphoreType.DMA,
            pltpu.SemaphoreType.DMA,
            pltpu.SemaphoreType.DMA,
            pltpu.SemaphoreType.DMA,
        ],
        compiler_params=pltpu.CompilerParams(
            use_tc_tiling_on_sc=True, needs_layout_passes=False),
    )
    def sc_kernel(x_hbm, enc_hbm, out_hbm, enc_v, xs_v, ld0, ld1, st0, st1):
        wid = lax.axis_index("s") * _NC + lax.axis_index("c")
        base = wid * CHUNK
        ld = (ld0, ld1)
        st = (st0, st1)

        def start_load(t):
            p = t % 2
            r0 = base + t * R
            pltpu.async_copy(enc_hbm.at[pl.ds(r0, R), :], enc_v.at[p], ld[p])
            pltpu.async_copy(x_hbm.at[:, pl.ds(r0, R), :], xs_v.at[p], ld[p])

        def wait_load(t):
            p = t % 2
            r0 = base + t * R
            pltpu.make_async_copy(
                enc_hbm.at[pl.ds(r0, R), :], enc_v.at[p], ld[p]).wait()
            pltpu.make_async_copy(
                x_hbm.at[:, pl.ds(r0, R), :], xs_v.at[p], ld[p]).wait()

        def wait_store(t):
            p = t % 2
            r0 = base + t * R
            for b in range(B):
                pltpu.make_async_copy(
                    xs_v.at[p, b], out_hbm.at[b, pl.ds(r0, R), :],
                    st[p]).wait()

        def process(t):
            p = t % 2
            r0 = base + t * R
            zero = jnp.zeros((_L,), jnp.float32)

            @plsc.parallel_loop(0, RE, step=_L, unroll=4,
                                carry=(zero, zero, zero, zero))
            def accs(o, carry):
                r = o >> 10
                c = pl.multiple_of(o & (E - 1), _L)
                return tuple(
                    jnp.where(xs_v[p, b, r, pl.ds(c, _L)] == 0.0, 1.0, a)
                    for b, a in enumerate(carry)
                )

            hit = lax.reduce_max(
                jnp.maximum(jnp.maximum(accs[0], accs[1]),
                            jnp.maximum(accs[2], accs[3])), (0,))

            @pl.when(hit == 0.0)
            def _fast():
                for b in range(B):
                    pltpu.async_copy(
                        enc_v.at[p], out_hbm.at[b, pl.ds(r0, R), :], st[p])

            @pl.when(hit != 0.0)
            def _slow():
                @plsc.parallel_loop(0, RE, step=_L, unroll=4)
                def _(o):
                    r = o >> 10
                    c = pl.multiple_of(o & (E - 1), _L)
                    e = enc_v[p, r, pl.ds(c, _L)]
                    for b in range(B):
                        xv = xs_v[p, b, r, pl.ds(c, _L)]
                        xs_v[p, b, r, pl.ds(c, _L)] = jnp.where(
                            xv == 0.0, xv, e)

                for b in range(B):
                    pltpu.async_copy(
                        xs_v.at[p, b], out_hbm.at[b, pl.ds(r0, R), :], st[p])

        start_load(0)
        for t in range(NT):
            if t + 1 < NT:
                if t >= 1:
                    wait_store(t - 1)
                start_load(t + 1)
            wait_load(t)
            process(t)
        wait_store(NT - 2)
        wait_store(NT - 1)

    return sc_kernel(x, enc)


def kernel(x):
    B, S, E = x.shape
    enc = _enc_table(S, E)
    return _sc_call(x, enc, B, S, E)


# speedup vs baseline: 3.9614x; 1.2146x over previous
import functools
import numpy as np
import jax
import jax.numpy as jnp
from jax import lax
from jax.experimental import pallas as pl
from jax.experimental.pallas import tpu as pltpu
from jax.experimental.pallas import tpu_sc as plsc


def _enc_table(S, E):
    pos = np.arange(S, dtype=np.float64)[:, None]
    i = np.arange(E, dtype=np.float64)[None, :]
    angle = pos / np.power(10000.0, (i - np.mod(i, 2)) / E)
    enc = np.array(angle)
    enc[:, 0::2] = np.sin(angle[:, 0::2])
    enc[:, 1::2] = np.cos(angle[:, 1::2])
    return jnp.asarray(enc, dtype=jnp.float32)


_NC, _NS, _L = 2, 16, 16
_NW = _NC * _NS
_R = 8


def _sc_scan(x, B, S, E):
    CHUNK = S // _NW
    NT = CHUNK // _R

    mesh = plsc.VectorSubcoreMesh(core_axis_name="c", subcore_axis_name="s")

    @functools.partial(
        pl.kernel,
        out_type=jax.ShapeDtypeStruct((_NW, NT), jnp.float32),
        mesh=mesh,
        scratch_types=[
            pltpu.VMEM((2, B, _R, E), jnp.float32),
            pltpu.VMEM((NT,), jnp.float32),
            pltpu.SemaphoreType.DMA,
            pltpu.SemaphoreType.DMA,
        ],
        compiler_params=pltpu.CompilerParams(
            use_tc_tiling_on_sc=True, needs_layout_passes=False),
    )
    def scan_kernel(x_hbm, flags_hbm, xs_v, fl_v, ld0, ld1):
        wid = lax.axis_index("s") * _NC + lax.axis_index("c")
        base = wid * CHUNK
        ld = (ld0, ld1)
        RE = _R * E

        def start_load(t):
            p = t % 2
            r0 = base + t * _R
            pltpu.async_copy(x_hbm.at[:, pl.ds(r0, _R), :], xs_v.at[p], ld[p])

        def wait_load(t):
            p = t % 2
            r0 = base + t * _R
            pltpu.make_async_copy(
                x_hbm.at[:, pl.ds(r0, _R), :], xs_v.at[p], ld[p]).wait()

        lanes = lax.iota(jnp.int32, _L)
        hitvec = jnp.zeros((_L,), jnp.float32)
        zero = jnp.zeros((_L,), jnp.float32)

        start_load(0)
        for t in range(NT):
            if t + 1 < NT:
                start_load(t + 1)
            wait_load(t)
            p = t % 2

            @plsc.parallel_loop(0, RE, step=_L, unroll=4,
                                carry=(zero, zero, zero, zero))
            def accs(o, carry):
                r = o >> 10
                c = pl.multiple_of(o & (E - 1), _L)
                return tuple(
                    jnp.where(xs_v[p, b, r, pl.ds(c, _L)] == 0.0, 1.0, a)
                    for b, a in enumerate(carry)
                )

            hit = lax.reduce_max(
                jnp.maximum(jnp.maximum(accs[0], accs[1]),
                            jnp.maximum(accs[2], accs[3])), (0,))
            hitvec = jnp.where(lanes == t, hit, hitvec)

        fl_v[...] = hitvec
        pltpu.sync_copy(fl_v, flags_hbm.at[wid])

    return scan_kernel(x)


def _tc_broadcast(enc, B, S, E):
    BS = 256

    def body(enc_ref, o_ref):
        o_ref[...] = jnp.broadcast_to(enc_ref[...][None], (B, BS, E))

    return pl.pallas_call(
        body,
        grid=(S // BS,),
        in_specs=[pl.BlockSpec((BS, E), lambda s: (s, 0))],
        out_specs=pl.BlockSpec((B, BS, E), lambda s: (0, s, 0)),
        out_shape=jax.ShapeDtypeStruct((B, S, E), jnp.float32),
    )(enc)


def _tc_patch(flags, x, enc, out0, B, S, E):
    CHUNK = S // _NW
    NT = CHUNK // _R

    def body(fl_vec_ref, fl_ref, x_ref, enc_ref, out0_ref, o_ref,
             xb_ref, eb_ref, sem):
        glob = jnp.max(fl_vec_ref[...])

        @pl.when(glob > 0.0)
        def _():
            def w_loop(w, carry):
                def t_loop(t, carry2):
                    f = fl_ref[w, t]

                    @pl.when(f > 0.0)
                    def _patch():
                        r0 = w * CHUNK + t * _R
                        pltpu.make_async_copy(
                            enc_ref.at[pl.ds(r0, _R), :], eb_ref, sem).start()
                        pltpu.make_async_copy(
                            enc_ref.at[pl.ds(r0, _R), :], eb_ref, sem).wait()
                        for b in range(B):
                            pltpu.make_async_copy(
                                x_ref.at[b, pl.ds(r0, _R), :], xb_ref,
                                sem).start()
                            pltpu.make_async_copy(
                                x_ref.at[b, pl.ds(r0, _R), :], xb_ref,
                                sem).wait()
                            xv = xb_ref[...]
                            xb_ref[...] = jnp.where(
                                xv == 0.0, xv, eb_ref[...])
                            pltpu.make_async_copy(
                                xb_ref, o_ref.at[b, pl.ds(r0, _R), :],
                                sem).start()
                            pltpu.make_async_copy(
                                xb_ref, o_ref.at[b, pl.ds(r0, _R), :],
                                sem).wait()

                    return carry2

                return lax.fori_loop(0, NT, t_loop, carry)

            lax.fori_loop(0, _NW, w_loop, 0)

    return pl.pallas_call(
        body,
        in_specs=[
            pl.BlockSpec(memory_space=pltpu.VMEM),
            pl.BlockSpec(memory_space=pltpu.SMEM),
            pl.BlockSpec(memory_space=pl.ANY),
            pl.BlockSpec(memory_space=pl.ANY),
            pl.BlockSpec(memory_space=pl.ANY),
        ],
        out_specs=pl.BlockSpec(memory_space=pl.ANY),
        out_shape=jax.ShapeDtypeStruct((B, S, E), jnp.float32),
        scratch_shapes=[
            pltpu.VMEM((_R, E), jnp.float32),
            pltpu.VMEM((_R, E), jnp.float32),
            pltpu.SemaphoreType.DMA,
        ],
        input_output_aliases={4: 0},
    )(flags, flags, x, enc, out0)


@functools.partial(jax.jit, static_argnums=(2, 3, 4))
def _run(x, enc, B, S, E):
    flags = _sc_scan(x, B, S, E)
    out0 = _tc_broadcast(enc, B, S, E)
    return _tc_patch(flags, x, enc, out0, B, S, E)


def kernel(x):
    B, S, E = x.shape
    enc = _enc_table(S, E)
    return _run(x, enc, B, S, E)
